# baseline (speedup 1.0000x reference)
import jax
import jax.numpy as jnp
import numpy as np
from jax.experimental import pallas as pl
from jax.experimental.pallas import tpu as pltpu

HIDDEN = 2048
LATENT = 32768
K = 64

ENC_TILE = 1024
DEC_TILE = 1024
NBUF = 3

N_ENC = LATENT // ENC_TILE
N_DEC = LATENT // DEC_TILE

_INT_MIN = np.int32(-2147483648)


def _f32_key(x):
    b = jax.lax.bitcast_convert_type(x, jnp.int32)
    return jnp.where(b >= 0, b, jnp.bitwise_xor(jnp.bitwise_not(b), _INT_MIN))


def _avg_floor(lo, hi):
    return (lo & hi) + ((lo ^ hi) >> 1)


def _fused_kernel(x_ref, pb_ref, lb_ref, w_ref, dec_ref,
                  lat_ref, out_ref, acts_ref, gmax_ref, ring_ref, sems):
    i = pl.program_id(0)
    xm = x_ref[...] - pb_ref[...]
    tile = jax.lax.dot_general(
        xm, w_ref[...], (((1,), (1,)), ((), ())),
        preferred_element_type=jnp.float32)
    tile = tile + lb_ref[...]
    acts_ref[:, pl.ds(i * ENC_TILE, ENC_TILE)] = tile

    rows = tile.shape[0]
    tkey = _f32_key(tile)
    tgmax = jnp.max(tkey.reshape(rows, ENC_TILE // 128, 128), axis=1)

    @pl.when(i == 0)
    def _init_gmax():
        gmax_ref[...] = tgmax

    @pl.when(i > 0)
    def _upd_gmax():
        gmax_ref[...] = jnp.maximum(gmax_ref[...], tgmax)

    @pl.when(i == N_ENC - 1)
    def _epilogue():
        for j in range(NBUF):
            pltpu.make_async_copy(
                dec_ref.at[:, pl.ds(j * DEC_TILE, DEC_TILE)],
                ring_ref.at[j], sems.at[j]).start()

        acts = acts_ref[...]
        keys = _f32_key(acts)
        gmax = gmax_ref[...]

        rmax = jnp.max(gmax, axis=1, keepdims=True)
        lob = jnp.min(gmax, axis=1, keepdims=True)

        def val_cond(c):
            lo, hi = c
            return jnp.any(lo < hi)

        def val_step(c):
            lo, hi = c
            mid = _avg_floor(lo, hi)
            cnt = jnp.sum((keys > mid).astype(jnp.int32), axis=1,
                          keepdims=True)
            small = cnt < K
            return jnp.where(small, lo, mid + 1), jnp.where(small, mid, hi)

        def val_body(c):
            return val_step(val_step(c))

        thr, _ = jax.lax.while_loop(val_cond, val_body, (lob, rmax))

        mask_gt = keys > thr
        mask_eq = keys == thr
        n_gt = jnp.sum(mask_gt.astype(jnp.int32), axis=1, keepdims=True)
        n_eq = jnp.sum(mask_eq.astype(jnp.int32), axis=1, keepdims=True)
        need = K - n_gt

        idx = jax.lax.broadcasted_iota(jnp.int32, keys.shape, 1)

        def tie_break(_):
            def idx_body(_, c):
                lo, hi = c
                mid = (lo + hi) >> 1
                cnt = jnp.sum((mask_eq & (idx < mid)).astype(jnp.int32),
                              axis=1, keepdims=True)
                enough = cnt >= need
                return (jnp.where(enough, lo, mid + 1),
                        jnp.where(enough, mid, hi))

            li = jnp.zeros((rows, 1), jnp.int32)
            hi = jnp.full((rows, 1), LATENT, jnp.int32)
            _, jstar = jax.lax.fori_loop(0, 16, idx_body, (li, hi))
            return jstar

        jstar = jax.lax.cond(
            jnp.all(n_eq == need),
            lambda _: jnp.full((rows, 1), LATENT, jnp.int32),
            tie_break, operand=None)

        keep = mask_gt | (mask_eq & (idx < jstar))
        lat_ref[...] = jnp.where(keep, acts, 0.0)

        out_ref[...] = jnp.broadcast_to(pb_ref[...], (rows, HIDDEN))
        for j in range(N_DEC):
            slot = j % NBUF
            pltpu.make_async_copy(
                dec_ref.at[:, pl.ds(j * DEC_TILE, DEC_TILE)],
                ring_ref.at[slot], sems.at[slot]).wait()
            out_ref[...] += jax.lax.dot_general(
                lat_ref[:, pl.ds(j * DEC_TILE, DEC_TILE)],
                ring_ref[slot], (((1,), (1,)), ((), ())),
                preferred_element_type=jnp.float32)
            if j + NBUF < N_DEC:
                pltpu.make_async_copy(
                    dec_ref.at[:, pl.ds((j + NBUF) * DEC_TILE, DEC_TILE)],
                    ring_ref.at[slot], sems.at[slot]).start()


@jax.jit
def kernel(x, pre_bias, latent_bias, enc_W, dec_W):
    b = x.shape[0]
    x2 = x.reshape(b, HIDDEN)
    pb = pre_bias.reshape(1, HIDDEN)
    lb = latent_bias.reshape(1, LATENT)

    latents, x_hat = pl.pallas_call(
        _fused_kernel,
        grid=(N_ENC,),
        in_specs=[
            pl.BlockSpec((b, HIDDEN), lambda i: (0, 0)),
            pl.BlockSpec((1, HIDDEN), lambda i: (0, 0)),
            pl.BlockSpec((1, ENC_TILE), lambda i: (0, i)),
            pl.BlockSpec((ENC_TILE, HIDDEN), lambda i: (i, 0)),
            pl.BlockSpec(memory_space=pl.ANY),
        ],
        out_specs=[
            pl.BlockSpec((b, LATENT), lambda i: (0, 0)),
            pl.BlockSpec((b, HIDDEN), lambda i: (0, 0)),
        ],
        out_shape=[
            jax.ShapeDtypeStruct((b, LATENT), jnp.float32),
            jax.ShapeDtypeStruct((b, HIDDEN), jnp.float32),
        ],
        scratch_shapes=[
            pltpu.VMEM((b, LATENT), jnp.float32),
            pltpu.VMEM((b, 128), jnp.int32),
            pltpu.VMEM((NBUF, HIDDEN, DEC_TILE), jnp.float32),
            pltpu.SemaphoreType.DMA((NBUF,)),
        ],
    )(x2, pb, lb, enc_W, dec_W)

    return latents.reshape(b, 1, LATENT), x_hat.reshape(b, 1, HIDDEN)


# speedup vs baseline: 1.0003x; 1.0003x over previous
import jax
import jax.numpy as jnp
import numpy as np
from jax.experimental import pallas as pl
from jax.experimental.pallas import tpu as pltpu

HIDDEN = 2048
LATENT = 32768
K = 64

ENC_TILE = 1024
DEC_TILE = 1024
NBUF = 3

N_ENC = LATENT // ENC_TILE
N_DEC = LATENT // DEC_TILE
PRE_START = 24
PREFIX = PRE_START * ENC_TILE

_INT_MIN = np.int32(-2147483648)


def _f32_key(x):
    b = jax.lax.bitcast_convert_type(x, jnp.int32)
    return jnp.where(b >= 0, b, jnp.bitwise_xor(jnp.bitwise_not(b), _INT_MIN))


def _key_f32(k):
    b = jnp.where(k >= 0, k, jnp.bitwise_not(jnp.bitwise_xor(k, _INT_MIN)))
    return jax.lax.bitcast_convert_type(b, jnp.float32)


def _avg_floor(lo, hi):
    return (lo & hi) + ((lo ^ hi) >> 1)


def _count_step(keys, c):
    lo, hi = c
    mid = _avg_floor(lo, hi)
    cnt = jnp.sum((keys > mid).astype(jnp.int32), axis=1, keepdims=True)
    small = cnt < K
    return jnp.where(small, lo, mid + 1), jnp.where(small, mid, hi)


def _fused_kernel(x_ref, pb_ref, lb_ref, w_ref, dec_ref,
                  lat_ref, out_ref, keys_ref, gmax_ref, lo_ref, hi_ref,
                  ring_ref, sems):
    i = pl.program_id(0)
    xm = x_ref[...] - pb_ref[...]
    tile = jax.lax.dot_general(
        xm, w_ref[...], (((1,), (1,)), ((), ())),
        preferred_element_type=jnp.float32)
    tkey = _f32_key(tile + lb_ref[...])
    keys_ref[:, pl.ds(i * ENC_TILE, ENC_TILE)] = tkey

    rows = tile.shape[0]
    tgmax = jnp.max(tkey.reshape(rows, ENC_TILE // 128, 128), axis=1)

    @pl.when(i == PRE_START)
    def _init_prefix():
        g = gmax_ref[...]
        lo_ref[...] = jnp.min(g, axis=1, keepdims=True)
        hi_ref[...] = jnp.max(g, axis=1, keepdims=True)

    @pl.when(i == 0)
    def _init_gmax():
        gmax_ref[...] = tgmax

    @pl.when(i > 0)
    def _upd_gmax():
        gmax_ref[...] = jnp.maximum(gmax_ref[...], tgmax)

    @pl.when(i >= PRE_START)
    def _prefix_search():
        pkeys = keys_ref[:, :PREFIX]
        c = (lo_ref[...], hi_ref[...])
        c = _count_step(pkeys, _count_step(pkeys, c))
        lo_ref[...], hi_ref[...] = c

    @pl.when(i == N_ENC - 1)
    def _epilogue():
        for j in range(NBUF):
            pltpu.make_async_copy(
                dec_ref.at[:, pl.ds(j * DEC_TILE, DEC_TILE)],
                ring_ref.at[j], sems.at[j]).start()

        keys = keys_ref[...]
        gmax = gmax_ref[...]
        rmax = jnp.max(gmax, axis=1, keepdims=True)

        def val_cond(c):
            lo, hi = c
            return jnp.any(lo < hi)

        def val_body(c):
            return _count_step(keys, _count_step(keys, c))

        thr, _ = jax.lax.while_loop(val_cond, val_body,
                                    (lo_ref[...], rmax))

        mask_gt = keys > thr
        mask_eq = keys == thr
        n_gt = jnp.sum(mask_gt.astype(jnp.int32), axis=1, keepdims=True)
        n_eq = jnp.sum(mask_eq.astype(jnp.int32), axis=1, keepdims=True)
        need = K - n_gt

        idx = jax.lax.broadcasted_iota(jnp.int32, keys.shape, 1)

        def tie_break(_):
            def idx_body(_, c):
                lo, hi = c
                mid = (lo + hi) >> 1
                cnt = jnp.sum((mask_eq & (idx < mid)).astype(jnp.int32),
                              axis=1, keepdims=True)
                enough = cnt >= need
                return (jnp.where(enough, lo, mid + 1),
                        jnp.where(enough, mid, hi))

            li = jnp.zeros((rows, 1), jnp.int32)
            hi = jnp.full((rows, 1), LATENT, jnp.int32)
            _, jstar = jax.lax.fori_loop(0, 16, idx_body, (li, hi))
            return jstar

        jstar = jax.lax.cond(
            jnp.all(n_eq == need),
            lambda _: jnp.full((rows, 1), LATENT, jnp.int32),
            tie_break, operand=None)

        keep = mask_gt | (mask_eq & (idx < jstar))
        lat_ref[...] = jnp.where(keep, _key_f32(keys), 0.0)

        out_ref[...] = jnp.broadcast_to(pb_ref[...], (rows, HIDDEN))
        for j in range(N_DEC):
            slot = j % NBUF
            pltpu.make_async_copy(
                dec_ref.at[:, pl.ds(j * DEC_TILE, DEC_TILE)],
                ring_ref.at[slot], sems.at[slot]).wait()
            out_ref[...] += jax.lax.dot_general(
                lat_ref[:, pl.ds(j * DEC_TILE, DEC_TILE)],
                ring_ref[slot], (((1,), (1,)), ((), ())),
                preferred_element_type=jnp.float32)
            if j + NBUF < N_DEC:
                pltpu.make_async_copy(
                    dec_ref.at[:, pl.ds((j + NBUF) * DEC_TILE, DEC_TILE)],
                    ring_ref.at[slot], sems.at[slot]).start()


@jax.jit
def kernel(x, pre_bias, latent_bias, enc_W, dec_W):
    b = x.shape[0]
    x2 = x.reshape(b, HIDDEN)
    pb = pre_bias.reshape(1, HIDDEN)
    lb = latent_bias.reshape(1, LATENT)

    latents, x_hat = pl.pallas_call(
        _fused_kernel,
        grid=(N_ENC,),
        in_specs=[
            pl.BlockSpec((b, HIDDEN), lambda i: (0, 0)),
            pl.BlockSpec((1, HIDDEN), lambda i: (0, 0)),
            pl.BlockSpec((1, ENC_TILE), lambda i: (0, i)),
            pl.BlockSpec((ENC_TILE, HIDDEN), lambda i: (i, 0)),
            pl.BlockSpec(memory_space=pl.ANY),
        ],
        out_specs=[
            pl.BlockSpec((b, LATENT), lambda i: (0, 0)),
            pl.BlockSpec((b, HIDDEN), lambda i: (0, 0)),
        ],
        out_shape=[
            jax.ShapeDtypeStruct((b, LATENT), jnp.float32),
            jax.ShapeDtypeStruct((b, HIDDEN), jnp.float32),
        ],
        scratch_shapes=[
            pltpu.VMEM((b, LATENT), jnp.int32),
            pltpu.VMEM((b, 128), jnp.int32),
            pltpu.VMEM((b, 1), jnp.int32),
            pltpu.VMEM((b, 1), jnp.int32),
            pltpu.VMEM((NBUF, HIDDEN, DEC_TILE), jnp.float32),
            pltpu.SemaphoreType.DMA((NBUF,)),
        ],
    )(x2, pb, lb, enc_W, dec_W)

    return latents.reshape(b, 1, LATENT), x_hat.reshape(b, 1, HIDDEN)


# speedup vs baseline: 1.1651x; 1.1647x over previous
import jax
import jax.numpy as jnp
import numpy as np
from jax.experimental import pallas as pl
from jax.experimental.pallas import tpu as pltpu

HIDDEN = 2048
LATENT = 32768
K = 64

ENC_TILE = 1024
DEC_TILE = 1024
NBUF = 3

N_ENC = LATENT // ENC_TILE
N_DEC = LATENT // DEC_TILE
PRE_START = 24
PREFIX = PRE_START * ENC_TILE

_INT_MIN = np.int32(-2147483648)


def _f32_key(x):
    b = jax.lax.bitcast_convert_type(x, jnp.int32)
    return jnp.where(b >= 0, b, jnp.bitwise_xor(jnp.bitwise_not(b), _INT_MIN))


def _key_f32(k):
    b = jnp.where(k >= 0, k, jnp.bitwise_not(jnp.bitwise_xor(k, _INT_MIN)))
    return jax.lax.bitcast_convert_type(b, jnp.float32)


def _avg_floor(lo, hi):
    return (lo & hi) + ((lo ^ hi) >> 1)


def _count_step(keys, c):
    lo, hi = c
    mid = _avg_floor(lo, hi)
    cnt = jnp.sum((keys > mid).astype(jnp.int32), axis=1, keepdims=True)
    small = cnt < K
    return jnp.where(small, lo, mid + 1), jnp.where(small, mid, hi)


def _fused_kernel(x_ref, pb_ref, lb_ref, w_ref, dec_ref,
                  lat_ref, out_ref, keys_ref, gmax_ref, ring_ref, sems):
    i = pl.program_id(0)
    xm = x_ref[...] - pb_ref[...]
    tile = jax.lax.dot_general(
        xm, w_ref[...], (((1,), (1,)), ((), ())),
        preferred_element_type=jnp.float32)
    tkey = _f32_key(tile + lb_ref[...])
    keys_ref[:, pl.ds(i * ENC_TILE, ENC_TILE)] = tkey

    rows = tile.shape[0]
    tgmax = jnp.max(tkey.reshape(rows, ENC_TILE // 128, 128), axis=1)

    @pl.when(i == 0)
    def _init_gmax():
        gmax_ref[...] = tgmax

    @pl.when(i > 0)
    def _upd_gmax():
        gmax_ref[...] = jnp.maximum(gmax_ref[...], tgmax)

    @pl.when(i == N_ENC - 1)
    def _epilogue():
        for j in range(NBUF):
            pltpu.make_async_copy(
                dec_ref.at[:, pl.ds(j * DEC_TILE, DEC_TILE)],
                ring_ref.at[j], sems.at[j]).start()

        keys = keys_ref[...]
        gmax = gmax_ref[...]
        rmax = jnp.max(gmax, axis=1, keepdims=True)
        lob = jnp.min(gmax, axis=1, keepdims=True)

        def val_cond(c):
            lo, hi = c
            return jnp.any(lo < hi)

        def val_body(c):
            return _count_step(keys, c)

        thr, _ = jax.lax.while_loop(val_cond, val_body, (lob, rmax))

        mask_gt = keys > thr
        mask_eq = keys == thr
        n_gt = jnp.sum(mask_gt.astype(jnp.int32), axis=1, keepdims=True)
        n_eq = jnp.sum(mask_eq.astype(jnp.int32), axis=1, keepdims=True)
        need = K - n_gt

        idx = jax.lax.broadcasted_iota(jnp.int32, keys.shape, 1)

        def tie_break(_):
            def idx_body(_, c):
                lo, hi = c
                mid = (lo + hi) >> 1
                cnt = jnp.sum((mask_eq & (idx < mid)).astype(jnp.int32),
                              axis=1, keepdims=True)
                enough = cnt >= need
                return (jnp.where(enough, lo, mid + 1),
                        jnp.where(enough, mid, hi))

            li = jnp.zeros((rows, 1), jnp.int32)
            hi = jnp.full((rows, 1), LATENT, jnp.int32)
            _, jstar = jax.lax.fori_loop(0, 16, idx_body, (li, hi))
            return jstar

        jstar = jax.lax.cond(
            jnp.all(n_eq == need),
            lambda _: jnp.full((rows, 1), LATENT, jnp.int32),
            tie_break, operand=None)

        keep = mask_gt | (mask_eq & (idx < jstar))
        lat_ref[...] = jnp.where(keep, _key_f32(keys), 0.0)

        out_ref[...] = jnp.broadcast_to(pb_ref[...], (rows, HIDDEN))
        for j in range(N_DEC):
            slot = j % NBUF
            pltpu.make_async_copy(
                dec_ref.at[:, pl.ds(j * DEC_TILE, DEC_TILE)],
                ring_ref.at[slot], sems.at[slot]).wait()
            out_ref[...] += jax.lax.dot_general(
                lat_ref[:, pl.ds(j * DEC_TILE, DEC_TILE)],
                ring_ref[slot], (((1,), (1,)), ((), ())),
                preferred_element_type=jnp.float32)
            if j + NBUF < N_DEC:
                pltpu.make_async_copy(
                    dec_ref.at[:, pl.ds((j + NBUF) * DEC_TILE, DEC_TILE)],
                    ring_ref.at[slot], sems.at[slot]).start()


@jax.jit
def kernel(x, pre_bias, latent_bias, enc_W, dec_W):
    b = x.shape[0]
    x2 = x.reshape(b, HIDDEN)
    pb = pre_bias.reshape(1, HIDDEN)
    lb = latent_bias.reshape(1, LATENT)

    latents, x_hat = pl.pallas_call(
        _fused_kernel,
        grid=(N_ENC,),
        in_specs=[
            pl.BlockSpec((b, HIDDEN), lambda i: (0, 0)),
            pl.BlockSpec((1, HIDDEN), lambda i: (0, 0)),
            pl.BlockSpec((1, ENC_TILE), lambda i: (0, i)),
            pl.BlockSpec((ENC_TILE, HIDDEN), lambda i: (i, 0)),
            pl.BlockSpec(memory_space=pl.ANY),
        ],
        out_specs=[
            pl.BlockSpec((b, LATENT), lambda i: (0, 0)),
            pl.BlockSpec((b, HIDDEN), lambda i: (0, 0)),
        ],
        out_shape=[
            jax.ShapeDtypeStruct((b, LATENT), jnp.float32),
            jax.ShapeDtypeStruct((b, HIDDEN), jnp.float32),
        ],
        scratch_shapes=[
            pltpu.VMEM((b, LATENT), jnp.int32),
            pltpu.VMEM((b, 128), jnp.int32),
            pltpu.VMEM((NBUF, HIDDEN, DEC_TILE), jnp.float32),
            pltpu.SemaphoreType.DMA((NBUF,)),
        ],
    )(x2, pb, lb, enc_W, dec_W)

    return latents.reshape(b, 1, LATENT), x_hat.reshape(b, 1, HIDDEN)
